# baseline (speedup 1.0000x reference)
import functools

import jax
import jax.numpy as jnp
from jax import lax
from jax.experimental import pallas as pl
from jax.experimental.pallas import tpu as pltpu
from jax.experimental.pallas import tpu_sc as plsc

_N_E = 8192
_E_DIM = 256
_BETA = 0.25

_BM = 512
_LANES = 128


def _dist_argmin_body(zf_ref, et2_ref, idx_ref, loss_ref, cnb_ref):
    i = pl.program_id(0)
    zfb = zf_ref[...]

    @pl.when(i == 0)
    def _():
        e2 = et2_ref[...]
        cn = jnp.sum(e2 * e2, axis=0, keepdims=True) * 0.25
        cnb_ref[...] = jnp.broadcast_to(cn, (8, _N_E))
        loss_ref[...] = jnp.zeros((1, 1), jnp.float32)

    m2 = jnp.dot(zfb, et2_ref[...],
                 preferred_element_type=jnp.float32)
    rn = jnp.sum(zfb * zfb, axis=1, keepdims=True)
    rnb = jnp.broadcast_to(rn, (_BM, _LANES))
    cnb = cnb_ref[...]

    n_slices = _N_E // _LANES
    rm_parts, rc_parts = [], []
    for g in range(_BM // 8):
        rg = slice(g * 8, (g + 1) * 8)
        rnb_g = rnb[rg, :]
        runmin = jnp.full((8, _LANES), jnp.float32(3e38))
        runc = jnp.zeros((8, _LANES), jnp.float32)
        for c in range(n_slices):
            sl = slice(c * _LANES, (c + 1) * _LANES)
            d = (rnb_g + cnb[:, sl]) + m2[rg, sl]
            lt = d < runmin
            runmin = jnp.where(lt, d, runmin)
            runc = jnp.where(lt, jnp.float32(c), runc)
        rm_parts.append(runmin)
        rc_parts.append(runc)

    rm = jnp.concatenate(rm_parts, axis=0)
    rc = jnp.concatenate(rc_parts, axis=0)
    lanef = lax.broadcasted_iota(
        jnp.int32, (_BM, _LANES), 1).astype(jnp.float32)
    colf = rc * jnp.float32(_LANES) + lanef
    minval = jnp.min(rm, axis=1, keepdims=True)
    idxf = jnp.min(jnp.where(rm == minval, colf, jnp.float32(3e38)), axis=1)
    idx_ref[0, :] = idxf.astype(jnp.int32)
    loss_ref[...] += jnp.sum(minval).reshape(1, 1)


def _dist_argmin(zf, et2):
    n_blocks = zf.shape[0] // _BM
    return pl.pallas_call(
        _dist_argmin_body,
        grid=(n_blocks,),
        in_specs=[
            pl.BlockSpec((_BM, _E_DIM), lambda i: (i, 0)),
            pl.BlockSpec((_E_DIM, _N_E), lambda i: (0, 0)),
        ],
        out_specs=[
            pl.BlockSpec((1, _BM), lambda i: (0, i)),
            pl.BlockSpec((1, 1), lambda i: (0, 0)),
        ],
        out_shape=[
            jax.ShapeDtypeStruct((1, zf.shape[0]), jnp.int32),
            jax.ShapeDtypeStruct((1, 1), jnp.float32),
        ],
        scratch_shapes=[pltpu.VMEM((8, _N_E), jnp.float32)],
    )(zf, et2)


def _make_sc_gather(B, D):
    info = plsc.get_sparse_core_info()
    nw = info.num_cores * info.num_subcores
    b_per_w = B // nw
    mesh = plsc.VectorSubcoreMesh(core_axis_name="c", subcore_axis_name="s")

    @functools.partial(
        pl.kernel, mesh=mesh,
        out_type=jax.ShapeDtypeStruct((B, D), jnp.float32),
        scratch_types=[
            pltpu.VMEM((b_per_w,), jnp.int32),
            pltpu.VMEM((b_per_w, D), jnp.float32),
            pltpu.SemaphoreType.DMA,
        ],
    )
    def gather(table_hbm, idx_hbm, out_hbm, idx_v, rows_v, sem):
        wid = lax.axis_index("s") * info.num_cores + lax.axis_index("c")
        base = wid * b_per_w
        pltpu.sync_copy(idx_hbm.at[pl.ds(base, b_per_w)], idx_v)
        pltpu.async_copy(table_hbm.at[idx_v], rows_v, sem).wait()
        pltpu.sync_copy(rows_v, out_hbm.at[pl.ds(base, b_per_w)])

    return gather


def kernel(z, emb):
    b, c, h, w = z.shape
    zp = jnp.transpose(z, (0, 2, 3, 1))
    zf = zp.reshape(-1, _E_DIM)
    nrows = zf.shape[0]
    et2 = -2.0 * emb.T

    n_chunks = 2
    rows_c = nrows // n_chunks
    gather = _make_sc_gather(rows_c, _E_DIM)
    idx_parts, zq_parts, loss_parts = [], [], []
    for ci in range(n_chunks):
        zf_c = lax.slice_in_dim(zf, ci * rows_c, (ci + 1) * rows_c, axis=0)
        idx2d, loss_sum = _dist_argmin(zf_c, et2)
        idx_c = idx2d.reshape(rows_c)
        zq_c = gather(emb, idx_c)
        idx_parts.append(idx_c)
        zq_parts.append(zq_c.reshape(b // n_chunks, h, w, c).transpose(0, 3, 1, 2))
        loss_parts.append(loss_sum[0, 0])

    idx = jnp.concatenate(idx_parts)
    z_q = jnp.concatenate(zq_parts, axis=0)
    mean_sq = (loss_parts[0] + loss_parts[1]) / jnp.float32(nrows * _E_DIM)
    loss = mean_sq + _BETA * mean_sq
    return (z_q, loss, idx)


# speedup vs baseline: 1.1964x; 1.1964x over previous
import functools

import jax
import jax.numpy as jnp
from jax import lax
from jax.experimental import pallas as pl
from jax.experimental.pallas import tpu as pltpu
from jax.experimental.pallas import tpu_sc as plsc

_N_E = 8192
_E_DIM = 256
_BETA = 0.25

_BM = 512
_LANES = 128


def _dist_argmin_body(z_ref, emb_ref, idx_ref, loss_ref, cnb_ref, et_ref):
    i = pl.program_id(0)

    @pl.when(i == 0)
    def _():
        et = emb_ref[...].T
        et_ref[...] = et
        cn = jnp.sum(et * et, axis=0, keepdims=True)
        cnb_ref[...] = jnp.broadcast_to(cn, (8, _N_E))
        loss_ref[...] = jnp.zeros((1, 1), jnp.float32)

    zfb = z_ref[0].T
    m2 = jnp.dot(zfb * jnp.float32(-2.0), et_ref[...],
                 preferred_element_type=jnp.float32)
    rn = jnp.sum(zfb * zfb, axis=1, keepdims=True)
    rnb = jnp.broadcast_to(rn, (_BM, _LANES))
    cnb = cnb_ref[...]

    n_slices = _N_E // _LANES
    rm_parts, rc_parts = [], []
    for g in range(_BM // 8):
        rg = slice(g * 8, (g + 1) * 8)
        rnb_g = rnb[rg, :]
        runmin = jnp.full((8, _LANES), jnp.float32(3e38))
        runc = jnp.zeros((8, _LANES), jnp.float32)
        for c in range(n_slices):
            sl = slice(c * _LANES, (c + 1) * _LANES)
            d = (rnb_g + cnb[:, sl]) + m2[rg, sl]
            lt = d < runmin
            runmin = jnp.where(lt, d, runmin)
            runc = jnp.where(lt, jnp.float32(c), runc)
        rm_parts.append(runmin)
        rc_parts.append(runc)

    rm = jnp.concatenate(rm_parts, axis=0)
    rc = jnp.concatenate(rc_parts, axis=0)
    lanef = lax.broadcasted_iota(
        jnp.int32, (_BM, _LANES), 1).astype(jnp.float32)
    colf = rc * jnp.float32(_LANES) + lanef
    minval = jnp.min(rm, axis=1, keepdims=True)
    idxf = jnp.min(jnp.where(rm == minval, colf, jnp.float32(3e38)), axis=1)
    idx_ref[0, :] = idxf.astype(jnp.int32)
    loss_ref[...] += jnp.sum(minval).reshape(1, 1)


def _dist_argmin(z3, emb):
    b_sz, _, hw = z3.shape
    nrows = b_sz * hw
    n_blocks = nrows // _BM
    per_b = hw // _BM
    return pl.pallas_call(
        _dist_argmin_body,
        grid=(n_blocks,),
        in_specs=[
            pl.BlockSpec((1, _E_DIM, _BM),
                         lambda i: (i // per_b, 0, i % per_b)),
            pl.BlockSpec((_N_E, _E_DIM), lambda i: (0, 0)),
        ],
        out_specs=[
            pl.BlockSpec((1, _BM), lambda i: (0, i)),
            pl.BlockSpec((1, 1), lambda i: (0, 0)),
        ],
        out_shape=[
            jax.ShapeDtypeStruct((1, nrows), jnp.int32),
            jax.ShapeDtypeStruct((1, 1), jnp.float32),
        ],
        scratch_shapes=[
            pltpu.VMEM((8, _N_E), jnp.float32),
            pltpu.VMEM((_E_DIM, _N_E), jnp.float32),
        ],
    )(z3, emb)


def _make_sc_gather(B, D):
    info = plsc.get_sparse_core_info()
    nw = info.num_cores * info.num_subcores
    b_per_w = B // nw
    mesh = plsc.VectorSubcoreMesh(core_axis_name="c", subcore_axis_name="s")

    @functools.partial(
        pl.kernel, mesh=mesh,
        out_type=jax.ShapeDtypeStruct((B, D), jnp.float32),
        scratch_types=[
            pltpu.VMEM((b_per_w,), jnp.int32),
            pltpu.VMEM((b_per_w, D), jnp.float32),
            pltpu.SemaphoreType.DMA,
        ],
    )
    def gather(table_hbm, idx_hbm, out_hbm, idx_v, rows_v, sem):
        wid = lax.axis_index("s") * info.num_cores + lax.axis_index("c")
        base = wid * b_per_w
        pltpu.sync_copy(idx_hbm.at[pl.ds(base, b_per_w)], idx_v)
        pltpu.async_copy(table_hbm.at[idx_v], rows_v, sem).wait()
        pltpu.sync_copy(rows_v, out_hbm.at[pl.ds(base, b_per_w)])

    return gather


def kernel(z, emb):
    b, c, h, w = z.shape
    nrows = b * h * w

    idx2d, loss_sum = _dist_argmin(z.reshape(b, c, h * w), emb)
    idx = idx2d.reshape(nrows)

    z_q_flat = _make_sc_gather(nrows, _E_DIM)(emb, idx)
    z_q = z_q_flat.reshape(b, h, w, c).transpose(0, 3, 1, 2)

    mean_sq = loss_sum[0, 0] / jnp.float32(nrows * _E_DIM)
    loss = mean_sq + _BETA * mean_sq
    return (z_q, loss, idx)


# speedup vs baseline: 1.2051x; 1.0073x over previous
import functools

import jax
import jax.numpy as jnp
from jax import lax
from jax.experimental import pallas as pl
from jax.experimental.pallas import tpu as pltpu
from jax.experimental.pallas import tpu_sc as plsc

_N_E = 8192
_E_DIM = 256
_BETA = 0.25

_BM = 512
_LANES = 128
_MMCOLS = 2048


def _dist_argmin_body(z_ref, emb_ref, idx_ref, loss_ref, cnb_ref, et_ref):
    i = pl.program_id(0)

    @pl.when(i == 0)
    def _():
        et = emb_ref[...].T
        et_ref[...] = et
        cn = jnp.sum(et * et, axis=0, keepdims=True)
        cnb_ref[...] = jnp.broadcast_to(cn, (8, _N_E))
        loss_ref[...] = jnp.zeros((1, 1), jnp.float32)

    zfb = z_ref[0].T
    zneg = zfb * jnp.float32(-2.0)
    n_mm = _N_E // _MMCOLS
    m2s = [
        jnp.dot(zneg, et_ref[:, pl.ds(mc * _MMCOLS, _MMCOLS)],
                preferred_element_type=jnp.float32)
        for mc in range(n_mm)
    ]
    rn = jnp.sum(zfb * zfb, axis=1, keepdims=True)
    rnb = jnp.broadcast_to(rn, (_BM, _LANES))
    cnb = cnb_ref[...]

    sl_per_mm = _MMCOLS // _LANES
    rm_parts, rc_parts = [], []
    for g in range(_BM // 8):
        rg = slice(g * 8, (g + 1) * 8)
        rnb_g = rnb[rg, :]
        runmin = jnp.full((8, _LANES), jnp.float32(3e38))
        runc = jnp.zeros((8, _LANES), jnp.float32)
        for mc in range(n_mm):
            for c in range(sl_per_mm):
                sl = slice(c * _LANES, (c + 1) * _LANES)
                cg = slice(mc * _MMCOLS + c * _LANES,
                           mc * _MMCOLS + (c + 1) * _LANES)
                d = (rnb_g + cnb[:, cg]) + m2s[mc][rg, sl]
                lt = d < runmin
                runmin = jnp.where(lt, d, runmin)
                runc = jnp.where(lt, jnp.float32(mc * sl_per_mm + c), runc)
        rm_parts.append(runmin)
        rc_parts.append(runc)

    rm = jnp.concatenate(rm_parts, axis=0)
    rc = jnp.concatenate(rc_parts, axis=0)
    lanef = lax.broadcasted_iota(
        jnp.int32, (_BM, _LANES), 1).astype(jnp.float32)
    colf = rc * jnp.float32(_LANES) + lanef
    minval = jnp.min(rm, axis=1, keepdims=True)
    idxf = jnp.min(jnp.where(rm == minval, colf, jnp.float32(3e38)), axis=1)
    idx_ref[0, :] = idxf.astype(jnp.int32)
    loss_ref[...] += jnp.sum(minval).reshape(1, 1)


def _dist_argmin(z3, emb):
    b_sz, _, hw = z3.shape
    nrows = b_sz * hw
    n_blocks = nrows // _BM
    per_b = hw // _BM
    return pl.pallas_call(
        _dist_argmin_body,
        grid=(n_blocks,),
        in_specs=[
            pl.BlockSpec((1, _E_DIM, _BM),
                         lambda i: (i // per_b, 0, i % per_b)),
            pl.BlockSpec((_N_E, _E_DIM), lambda i: (0, 0)),
        ],
        out_specs=[
            pl.BlockSpec((1, _BM), lambda i: (0, i)),
            pl.BlockSpec((1, 1), lambda i: (0, 0)),
        ],
        out_shape=[
            jax.ShapeDtypeStruct((1, nrows), jnp.int32),
            jax.ShapeDtypeStruct((1, 1), jnp.float32),
        ],
        scratch_shapes=[
            pltpu.VMEM((8, _N_E), jnp.float32),
            pltpu.VMEM((_E_DIM, _N_E), jnp.float32),
        ],
    )(z3, emb)


def _make_sc_gather(B, D):
    info = plsc.get_sparse_core_info()
    nw = info.num_cores * info.num_subcores
    b_per_w = B // nw
    mesh = plsc.VectorSubcoreMesh(core_axis_name="c", subcore_axis_name="s")

    @functools.partial(
        pl.kernel, mesh=mesh,
        out_type=jax.ShapeDtypeStruct((B, D), jnp.float32),
        scratch_types=[
            pltpu.VMEM((b_per_w,), jnp.int32),
            pltpu.VMEM((b_per_w, D), jnp.float32),
            pltpu.SemaphoreType.DMA,
        ],
    )
    def gather(table_hbm, idx_hbm, out_hbm, idx_v, rows_v, sem):
        wid = lax.axis_index("s") * info.num_cores + lax.axis_index("c")
        base = wid * b_per_w
        pltpu.sync_copy(idx_hbm.at[pl.ds(base, b_per_w)], idx_v)
        pltpu.async_copy(table_hbm.at[idx_v], rows_v, sem).wait()
        pltpu.sync_copy(rows_v, out_hbm.at[pl.ds(base, b_per_w)])

    return gather


def kernel(z, emb):
    b, c, h, w = z.shape
    nrows = b * h * w

    idx2d, loss_sum = _dist_argmin(z.reshape(b, c, h * w), emb)
    idx = idx2d.reshape(nrows)

    z_q_flat = _make_sc_gather(nrows, _E_DIM)(emb, idx)
    z_q = z_q_flat.reshape(b, h, w, c).transpose(0, 3, 1, 2)

    mean_sq = loss_sum[0, 0] / jnp.float32(nrows * _E_DIM)
    loss = mean_sq + _BETA * mean_sq
    return (z_q, loss, idx)
